# Initial kernel scaffold; baseline (speedup 1.0000x reference)
import jax
import jax.numpy as jnp
from jax.experimental import pallas as pl


def kernel(x, edge_index, edge_attr, W0, b0, g0, be0, Wm, bm, gm, bem, W4, b4, g4, be4, Wf, bf):
    raise NotImplementedError("write your pallas kernel here")


# speedup vs baseline: 1.5203x; 1.5203x over previous
import functools
import jax
import jax.numpy as jnp
from jax import lax
from jax.experimental import pallas as pl
from jax.experimental.pallas import tpu as pltpu
from jax.experimental.pallas import tpu_sc as plsc

NN = 50000
NP = 50048
RT = NP // 16
F = 128
FG = 16
NFG = F // FG
E0 = 1600000
ET = E0 + NN
CH = 512
NCH = 101
PT = CH * NCH
EPAD = PT * 32
NB = 400
GN = NN // NB
ER = EPAD // 128
EPS = 1e-5

_mesh = lambda: plsc.VectorSubcoreMesh(core_axis_name="c", subcore_axis_name="s")


def _sc_gather_rows(table, idx):
    @functools.partial(
        pl.kernel,
        mesh=_mesh(),
        compiler_params=pltpu.CompilerParams(use_tc_tiling_on_sc=False),
        out_type=jax.ShapeDtypeStruct((EPAD, F), jnp.float32),
        scratch_types=[
            pltpu.VMEM((CH,), jnp.int32),
            pltpu.VMEM((CH, F), jnp.float32),
            pltpu.SemaphoreType.DMA,
        ],
    )
    def k(table_hbm, idx_hbm, out_hbm, idx_v, rows_v, sem):
        wid = lax.axis_index("s") * 2 + lax.axis_index("c")
        base = wid * PT

        def body(i, carry):
            off = base + i * CH
            pltpu.sync_copy(idx_hbm.at[pl.ds(off, CH)], idx_v)
            pltpu.async_copy(table_hbm.at[idx_v], rows_v, sem).wait()
            pltpu.sync_copy(rows_v, out_hbm.at[pl.ds(off, CH)])
            return carry

        lax.fori_loop(0, NCH, body, 0)

    return k(table, idx)


def _sc_scatter_add_rows(msg_g, idx, zeros_blk):
    @functools.partial(
        pl.kernel,
        mesh=_mesh(),
        compiler_params=pltpu.CompilerParams(use_tc_tiling_on_sc=False),
        out_type=jax.ShapeDtypeStruct((2 * NFG, NP, FG), jnp.float32),
        scratch_types=[
            pltpu.VMEM((CH,), jnp.int32),
            pltpu.VMEM((CH, FG), jnp.float32),
            pltpu.VMEM((RT, FG), jnp.float32),
            pltpu.VMEM_SHARED((NP, FG), jnp.float32),
        ],
    )
    def k(msg_hbm, idx_hbm, zb_hbm, out_hbm, idx_v, m_v, st_v, acc_sh):
        cid = lax.axis_index("c")
        sid = lax.axis_index("s")
        wid = sid * 2 + cid
        base = wid * PT
        r0 = sid * RT
        pltpu.sync_copy(zb_hbm, st_v)
        for fg in range(NFG):
            pltpu.sync_copy(st_v, acc_sh.at[pl.ds(r0, RT)])
            plsc.subcore_barrier()

            def body(i, carry):
                off = base + i * CH
                pltpu.sync_copy(idx_hbm.at[pl.ds(off, CH)], idx_v)
                pltpu.sync_copy(msg_hbm.at[fg, pl.ds(off, CH)], m_v)
                pltpu.sync_copy(m_v, acc_sh.at[idx_v], add=True)
                return carry

            lax.fori_loop(0, NCH, body, 0)
            plsc.subcore_barrier()
            pltpu.sync_copy(acc_sh.at[pl.ds(r0, RT)], st_v)
            pltpu.sync_copy(st_v, out_hbm.at[cid * NFG + fg, pl.ds(r0, RT)])
            pltpu.sync_copy(zb_hbm, st_v)
            plsc.subcore_barrier()

    return k(msg_g, idx, zeros_blk)


def _sc_scatter_add_scal(vals, idx, zeros_blk):
    @functools.partial(
        pl.kernel,
        mesh=_mesh(),
        compiler_params=pltpu.CompilerParams(use_tc_tiling_on_sc=False),
        out_type=jax.ShapeDtypeStruct((2 * NP,), jnp.float32),
        scratch_types=[
            pltpu.VMEM((CH,), jnp.int32),
            pltpu.VMEM((CH,), jnp.float32),
            pltpu.VMEM((RT,), jnp.float32),
            pltpu.VMEM_SHARED((NP,), jnp.float32),
        ],
    )
    def k(vals_hbm, idx_hbm, zb_hbm, out_hbm, idx_v, v_v, st_v, acc_sh):
        cid = lax.axis_index("c")
        sid = lax.axis_index("s")
        wid = sid * 2 + cid
        base = wid * PT
        r0 = sid * RT
        pltpu.sync_copy(zb_hbm, st_v)
        pltpu.sync_copy(st_v, acc_sh.at[pl.ds(r0, RT)])
        plsc.subcore_barrier()

        def body(i, carry):
            off = base + i * CH
            pltpu.sync_copy(idx_hbm.at[pl.ds(off, CH)], idx_v)
            pltpu.sync_copy(vals_hbm.at[pl.ds(off, CH)], v_v)
            pltpu.sync_copy(v_v, acc_sh.at[idx_v], add=True)
            return carry

        lax.fori_loop(0, NCH, body, 0)
        plsc.subcore_barrier()
        pltpu.sync_copy(acc_sh.at[pl.ds(r0, RT)], st_v)
        pltpu.sync_copy(st_v, out_hbm.at[pl.ds(cid * NP + r0, RT)])

    return k(vals, idx, zeros_blk)


def _tc_mm(x, w, b=None):
    M, K = x.shape
    Fo = w.shape[1]
    if b is None:
        def kern(x_ref, w_ref, o_ref):
            o_ref[...] = jnp.dot(x_ref[...], w_ref[...],
                                 preferred_element_type=jnp.float32)
        args = (x, w)
        in_specs = [pl.BlockSpec((NB, K), lambda i: (i, 0)),
                    pl.BlockSpec((K, Fo), lambda i: (0, 0))]
    else:
        def kern(x_ref, w_ref, b_ref, o_ref):
            o_ref[...] = jnp.dot(x_ref[...], w_ref[...],
                                 preferred_element_type=jnp.float32) + b_ref[...]
        args = (x, w, b.reshape(1, Fo))
        in_specs = [pl.BlockSpec((NB, K), lambda i: (i, 0)),
                    pl.BlockSpec((K, Fo), lambda i: (0, 0)),
                    pl.BlockSpec((1, Fo), lambda i: (0, 0))]
    return pl.pallas_call(
        kern, grid=(M // NB,), in_specs=in_specs,
        out_specs=pl.BlockSpec((NB, Fo), lambda i: (i, 0)),
        out_shape=jax.ShapeDtypeStruct((M, Fo), jnp.float32),
    )(*args)


def _tc_combine(p0, p1, dis2d, b, relu):
    def kern(a_ref, c_ref, d_ref, b_ref, o_ref):
        s = (a_ref[...] + c_ref[...]) * d_ref[...] + b_ref[...]
        o_ref[...] = jnp.maximum(s, 0.0) if relu else s
    return pl.pallas_call(
        kern, grid=(GN,),
        in_specs=[pl.BlockSpec((NB, F), lambda i: (i, 0)),
                  pl.BlockSpec((NB, F), lambda i: (i, 0)),
                  pl.BlockSpec((NB, 1), lambda i: (i, 0)),
                  pl.BlockSpec((1, F), lambda i: (0, 0))],
        out_specs=pl.BlockSpec((NB, F), lambda i: (i, 0)),
        out_shape=jax.ShapeDtypeStruct((NN, F), jnp.float32),
    )(p0, p1, dis2d, b.reshape(1, F))


def _tc_stats(h):
    def kern(h_ref, s_ref, ss_ref):
        i = pl.program_id(0)

        @pl.when(i == 0)
        def _():
            s_ref[...] = jnp.zeros_like(s_ref)
            ss_ref[...] = jnp.zeros_like(ss_ref)

        hb = h_ref[...]
        s_ref[...] += jnp.sum(hb, axis=0, keepdims=True)
        ss_ref[...] += jnp.sum(hb * hb, axis=0, keepdims=True)

    return pl.pallas_call(
        kern, grid=(GN,),
        in_specs=[pl.BlockSpec((NB, F), lambda i: (i, 0))],
        out_specs=[pl.BlockSpec((1, F), lambda i: (0, 0)),
                   pl.BlockSpec((1, F), lambda i: (0, 0))],
        out_shape=[jax.ShapeDtypeStruct((1, F), jnp.float32),
                   jax.ShapeDtypeStruct((1, F), jnp.float32)],
    )(h)


def _tc_bn(h, s, ss, g, be):
    def kern(h_ref, s_ref, ss_ref, g_ref, be_ref, o_ref):
        m = s_ref[...] / NN
        v = ss_ref[...] / NN - m * m
        o_ref[...] = (h_ref[...] - m) * lax.rsqrt(v + EPS) * g_ref[...] + be_ref[...]
    return pl.pallas_call(
        kern, grid=(GN,),
        in_specs=[pl.BlockSpec((NB, F), lambda i: (i, 0)),
                  pl.BlockSpec((1, F), lambda i: (0, 0)),
                  pl.BlockSpec((1, F), lambda i: (0, 0)),
                  pl.BlockSpec((1, F), lambda i: (0, 0)),
                  pl.BlockSpec((1, F), lambda i: (0, 0))],
        out_specs=pl.BlockSpec((NB, F), lambda i: (i, 0)),
        out_shape=jax.ShapeDtypeStruct((NN, F), jnp.float32),
    )(h, s, ss, g.reshape(1, F), be.reshape(1, F))


def _tc_dis(p0, p1):
    def kern(a_ref, b_ref, o_ref):
        d = a_ref[...] + b_ref[...]
        o_ref[...] = jnp.where(d > 0, lax.rsqrt(d), 0.0)
    return pl.pallas_call(
        kern,
        out_shape=jax.ShapeDtypeStruct((NN // 16, 16), jnp.float32),
    )(p0, p1)


def _tc_scale(g, w2d, rb):
    M = g.shape[0]
    def kern(g_ref, n_ref, o_ref):
        o_ref[...] = g_ref[...] * n_ref[...]
    return pl.pallas_call(
        kern, grid=(M // rb,),
        in_specs=[pl.BlockSpec((rb, F), lambda i: (i, 0)),
                  pl.BlockSpec((rb, 1), lambda i: (i, 0))],
        out_specs=pl.BlockSpec((rb, F), lambda i: (i, 0)),
        out_shape=jax.ShapeDtypeStruct((M, F), jnp.float32),
    )(g, w2d)


def kernel(x, edge_index, edge_attr, W0, b0, g0, be0, Wm, bm, gm, bem,
           W4, b4, g4, be4, Wf, bf):
    idt = jnp.int32
    loop = jnp.arange(NN, dtype=idt)
    padn = EPAD - ET
    row = jnp.concatenate([edge_index[0].astype(idt), loop,
                           jnp.zeros((padn,), idt)])
    col = jnp.concatenate([edge_index[1].astype(idt), loop,
                           jnp.zeros((padn,), idt)])
    ew = jnp.concatenate([edge_attr, jnp.ones((NN,), jnp.float32),
                          jnp.zeros((padn,), jnp.float32)])

    zb_s = jnp.zeros((RT,), jnp.float32)
    zb_v = jnp.zeros((RT, FG), jnp.float32)

    degp = _sc_scatter_add_scal(ew, col, zb_s).reshape(2, NP)
    dis = _tc_dis(degp[0, :NN].reshape(NN // 16, 16),
                  degp[1, :NN].reshape(NN // 16, 16)).reshape(NN, 1)
    ew2d = ew.reshape(EPAD, 1)

    def agg(z):
        zs = _tc_scale(z, dis, NB)
        g = _sc_gather_rows(zs, row)
        msg = _tc_scale(g, ew2d, 1024)
        msg_g = msg.reshape(EPAD, NFG, FG).transpose(1, 0, 2)
        p = _sc_scatter_add_rows(msg_g, col, zb_v).reshape(2, NFG, NP, FG)
        p0 = p[0, :, :NN, :].transpose(1, 0, 2).reshape(NN, F)
        p1 = p[1, :, :NN, :].transpose(1, 0, 2).reshape(NN, F)
        return p0, p1

    def pad_w(w):
        out = jnp.zeros((F, F), jnp.float32)
        return out.at[:w.shape[0], :w.shape[1]].set(w)

    def pad_v(v):
        return jnp.zeros((F,), jnp.float32).at[:v.shape[0]].set(v)

    xp = jnp.pad(x, ((0, 0), (0, F - x.shape[1])))
    W0p = jnp.zeros((F, F), jnp.float32).at[:8, :].set(W0)
    h = _tc_mm(xp, W0p)
    p0, p1 = agg(h)
    h = _tc_combine(p0, p1, dis, b0, relu=True)
    s, ss = _tc_stats(h)
    h = _tc_bn(h, s, ss, g0, be0)

    for i in range(3):
        z = _tc_mm(h, Wm[i])
        p0, p1 = agg(z)
        h = _tc_combine(p0, p1, dis, bm[i], relu=True)
        s, ss = _tc_stats(h)
        h = _tc_bn(h, s, ss, gm[i], bem[i])

    p0, p1 = agg(h)
    a4 = _tc_combine(p0, p1, dis, jnp.zeros((F,), jnp.float32),
                     relu=False)
    h4 = _tc_mm(a4, pad_w(W4), pad_v(b4))
    s, ss = _tc_stats(h4)
    h4 = _tc_bn(h4, s, ss, pad_v(g4), pad_v(be4))

    out = _tc_mm(h4, pad_w(Wf), pad_v(bf))
    return out[:, :8]
